# Initial kernel scaffold; baseline (speedup 1.0000x reference)
import jax
import jax.numpy as jnp
from jax.experimental import pallas as pl


def kernel(x, ptr):
    raise NotImplementedError("write your pallas kernel here")


# speedup vs baseline: 3310.1038x; 3310.1038x over previous
import dataclasses
import functools

import jax
import jax.numpy as jnp
from jax import lax
from jax.experimental import pallas as pl
from jax.experimental.pallas import tpu as pltpu
from jax.experimental.pallas import tpu_sc as plsc

NUM_GRAPHS = 1024
OUT_DIM = 1024
NUM_WORKERS = 32
GPW = NUM_GRAPHS // NUM_WORKERS
CHUNK = 8192
PTR_PAD = 1040
LANES = 16

_cp = pltpu.CompilerParams()
if "needs_layout_passes" in pltpu.CompilerParams.__dataclass_fields__:
    _cp = dataclasses.replace(_cp, needs_layout_passes=False)


@functools.partial(
    pl.kernel,
    compiler_params=_cp,
    out_type=jax.ShapeDtypeStruct((NUM_GRAPHS, OUT_DIM), jnp.float32),
    mesh=plsc.VectorSubcoreMesh(core_axis_name="c", subcore_axis_name="s"),
    scratch_types=[
        pltpu.VMEM((PTR_PAD,), jnp.int32),
        pltpu.VMEM((CHUNK + 8,), jnp.int32),
        pltpu.VMEM((GPW * OUT_DIM,), jnp.float32),
    ],
)
def _count_kernel(x_hbm, ptr_hbm, out_hbm, ptr_v, xbuf, hist):
    total = x_hbm.shape[0]
    wid = lax.axis_index("s") * 2 + lax.axis_index("c")
    g0 = wid * GPW

    zeros16 = jnp.zeros((LANES,), jnp.float32)
    ones16 = jnp.ones((LANES,), jnp.float32)
    iota16 = lax.iota(jnp.int32, LANES)

    @pl.loop(0, GPW * OUT_DIM, step=LANES)
    def _(i):
        hist[pl.ds(i, LANES)] = zeros16

    pltpu.sync_copy(ptr_hbm, ptr_v)

    @pl.loop(0, GPW)
    def _(g):
        pv = ptr_v[pl.ds(g0 + g, LANES)]
        seg_start = pv[0]
        seg_end = pv[1]
        count = seg_end - seg_start
        base_idx = g * OUT_DIM
        nchunks = (count + (CHUNK - 1)) // CHUNK

        @pl.loop(0, nchunks)
        def _(c):
            cs = seg_start + c * CHUNK
            a = jnp.minimum((cs // 8) * 8, total - (CHUNK + 8))
            off = cs - a
            pltpu.sync_copy(x_hbm.at[pl.ds(a, CHUNK + 8)], xbuf)
            nproc = jnp.minimum(count - c * CHUNK, CHUNK)
            nvec = (nproc + (LANES - 1)) // LANES

            @pl.loop(0, nvec)
            def _(v):
                colors = xbuf[pl.ds(off + v * LANES, LANES)]
                pos = v * LANES + iota16
                valid = (pos < nproc) & (colors < OUT_DIM) & (colors >= 0)
                idx = base_idx + jnp.clip(colors, 0, OUT_DIM - 1)
                plsc.addupdate_scatter(hist, [idx], ones16, mask=valid)

    @pl.loop(0, GPW)
    def _(g):
        pltpu.sync_copy(hist.at[pl.ds(g * OUT_DIM, OUT_DIM)],
                        out_hbm.at[g0 + g])


def kernel(x, ptr):
    x32 = x.astype(jnp.int32)
    ptr32 = ptr.astype(jnp.int32)
    pad = jnp.full((PTR_PAD - ptr32.shape[0],), x32.shape[0], jnp.int32)
    ptrp = jnp.concatenate([ptr32, pad])
    return _count_kernel(x32, ptrp)


# speedup vs baseline: 4719.5476x; 1.4258x over previous
import dataclasses
import functools

import jax
import jax.numpy as jnp
from jax import lax
from jax.experimental import pallas as pl
from jax.experimental.pallas import tpu as pltpu
from jax.experimental.pallas import tpu_sc as plsc

NUM_GRAPHS = 1024
OUT_DIM = 1024
NUM_WORKERS = 32
GPW = NUM_GRAPHS // NUM_WORKERS
CHUNK = 8192
XBUF = CHUNK + 24
PTR_PAD = 1040
LANES = 16
UNROLL = 4
HIST_WORDS = GPW * OUT_DIM + 2048

_cp = pltpu.CompilerParams()
if "needs_layout_passes" in pltpu.CompilerParams.__dataclass_fields__:
    _cp = dataclasses.replace(_cp, needs_layout_passes=False)


@functools.partial(
    pl.kernel,
    compiler_params=_cp,
    out_type=jax.ShapeDtypeStruct((NUM_GRAPHS, OUT_DIM), jnp.float32),
    mesh=plsc.VectorSubcoreMesh(core_axis_name="c", subcore_axis_name="s"),
    scratch_types=[
        pltpu.VMEM((PTR_PAD,), jnp.int32),
        pltpu.VMEM((XBUF,), jnp.int32),
        pltpu.VMEM((XBUF,), jnp.int32),
        pltpu.VMEM((HIST_WORDS,), jnp.float32),
        pltpu.SemaphoreType.DMA,
        pltpu.SemaphoreType.DMA,
        pltpu.SemaphoreType.DMA,
    ],
)
def _count_kernel(x_hbm, ptr_hbm, out_hbm, ptr_v, buf0, buf1, hist,
                  sem0, sem1, wsem):
    total = x_hbm.shape[0]
    wid = lax.axis_index("s") * 2 + lax.axis_index("c")
    g0 = wid * GPW

    zeros16 = jnp.zeros((LANES,), jnp.float32)
    ones16 = jnp.ones((LANES,), jnp.float32)
    iota16 = lax.iota(jnp.int32, LANES)
    udim = jnp.uint32(OUT_DIM)

    @pl.loop(0, GPW * OUT_DIM, step=4 * LANES)
    def _(i):
        for u in range(4):
            hist[pl.ds(i + u * LANES, LANES)] = zeros16

    pltpu.sync_copy(ptr_hbm, ptr_v)

    pw = ptr_v[pl.ds(g0, LANES)]
    wstart = pw[0]
    pe = ptr_v[pl.ds(g0 + GPW, LANES)]
    wend = pe[0]
    wn = wend - wstart
    base_a = (wstart // 8) * 8
    nch = (wn + (CHUNK - 1)) // CHUNK

    def dma_start(c, buf, sem):
        a = jnp.minimum(base_a + c * CHUNK, total - (CHUNK + 8))
        pltpu.async_copy(x_hbm.at[pl.ds(a, CHUNK + 8)], buf.at[pl.ds(0, CHUNK + 8)], sem)

    def dma_wait(buf, sem):
        pltpu.make_async_copy(x_hbm.at[pl.ds(0, CHUNK + 8)],
                              buf.at[pl.ds(0, CHUNK + 8)], sem).wait()

    def scat(colors, mask, bidx):
        idx = bidx + colors
        plsc.addupdate_scatter(hist, [idx], ones16, mask=mask)

    def process(c, buf, g):
        cs = wstart + c * CHUNK
        a = jnp.minimum(base_a + c * CHUNK, total - (CHUNK + 8))
        off = cs - a
        npc = jnp.minimum(wn - c * CHUNK, CHUNK)
        ce = cs + npc

        def piece_cond(st):
            p, _ = st
            return p < ce

        def piece(st):
            p, g = st
            pv = ptr_v[pl.ds(g0 + g + 1, LANES)]
            gend = pv[0]
            e = jnp.minimum(gend, ce)
            n = e - p
            bidx = g * OUT_DIM
            boff = off + (p - cs)
            nf4 = n // (UNROLL * LANES)

            @pl.loop(0, nf4)
            def _(v4):
                b = boff + v4 * (UNROLL * LANES)
                for u in range(UNROLL):
                    colors = buf[pl.ds(b + u * LANES, LANES)]
                    mask = plsc.bitcast(colors, jnp.uint32) < udim
                    scat(colors, mask, bidx)

            nfull = n // LANES

            @pl.loop(nf4 * UNROLL, nfull)
            def _(v):
                colors = buf[pl.ds(boff + v * LANES, LANES)]
                mask = plsc.bitcast(colors, jnp.uint32) < udim
                scat(colors, mask, bidx)

            rem = n - nfull * LANES

            @pl.when(rem > 0)
            def _():
                colors = buf[pl.ds(boff + nfull * LANES, LANES)]
                mask = (plsc.bitcast(colors, jnp.uint32) < udim) & (iota16 < rem)
                scat(colors, mask, bidx)

            g = jnp.where(gend <= ce, g + 1, g)
            return (e, g)

        _, g = lax.while_loop(piece_cond, piece, (p := cs, g))
        return g

    @pl.when(nch > 0)
    def _():
        dma_start(jnp.int32(0), buf0, sem0)

    def pair(i, g):
        c = 2 * i

        @pl.when(c + 1 < nch)
        def _():
            dma_start(c + 1, buf1, sem1)

        dma_wait(buf0, sem0)
        g = process(c, buf0, g)

        def second(g):
            @pl.when(c + 2 < nch)
            def _():
                dma_start(c + 2, buf0, sem0)

            dma_wait(buf1, sem1)
            return process(c + 1, buf1, g)

        return lax.cond(c + 1 < nch, second, lambda g: g, g)

    lax.fori_loop(0, (nch + 1) // 2, pair, jnp.int32(0))

    for g in range(GPW):
        pltpu.async_copy(hist.at[pl.ds(g * OUT_DIM, OUT_DIM)],
                         out_hbm.at[g0 + g], wsem)
    for g in range(GPW):
        pltpu.make_async_copy(hist.at[pl.ds(g * OUT_DIM, OUT_DIM)],
                              out_hbm.at[g0 + g], wsem).wait()


def kernel(x, ptr):
    x32 = x.astype(jnp.int32)
    ptr32 = ptr.astype(jnp.int32)
    pad = jnp.full((PTR_PAD - ptr32.shape[0],), x32.shape[0], jnp.int32)
    ptrp = jnp.concatenate([ptr32, pad])
    return _count_kernel(x32, ptrp)


# speedup vs baseline: 14123.8772x; 2.9926x over previous
import dataclasses
import functools

import jax
import jax.numpy as jnp
from jax import lax
from jax.experimental import pallas as pl
from jax.experimental.pallas import tpu as pltpu
from jax.experimental.pallas import tpu_sc as plsc

NUM_GRAPHS = 1024
OUT_DIM = 1024
NUM_WORKERS = 32
GPW = NUM_GRAPHS // NUM_WORKERS
CHUNK = 8192
XBUF = CHUNK + 24
PTR_PAD = 1040
LANES = 16
UNROLL = 4
HIST_WORDS = GPW * OUT_DIM + 2048

_cp = pltpu.CompilerParams()
if "needs_layout_passes" in pltpu.CompilerParams.__dataclass_fields__:
    _cp = dataclasses.replace(_cp, needs_layout_passes=False)


@functools.partial(
    pl.kernel,
    compiler_params=_cp,
    out_type=jax.ShapeDtypeStruct((NUM_GRAPHS, OUT_DIM), jnp.float32),
    mesh=plsc.VectorSubcoreMesh(core_axis_name="c", subcore_axis_name="s"),
    scratch_types=[
        pltpu.VMEM((PTR_PAD,), jnp.int32),
        pltpu.VMEM((XBUF,), jnp.int32),
        pltpu.VMEM((XBUF,), jnp.int32),
        pltpu.VMEM((HIST_WORDS,), jnp.float32),
        pltpu.SemaphoreType.DMA,
        pltpu.SemaphoreType.DMA,
        pltpu.SemaphoreType.DMA,
    ],
)
def _count_kernel(x_hbm, ptr_hbm, out_hbm, ptr_v, buf0, buf1, hist,
                  sem0, sem1, wsem):
    total = x_hbm.shape[0]
    wid = lax.axis_index("s") * 2 + lax.axis_index("c")
    g0 = wid * GPW

    zeros16 = jnp.zeros((LANES,), jnp.float32)
    ones16 = jnp.ones((LANES,), jnp.float32)
    iota16 = lax.iota(jnp.int32, LANES)
    udim = jnp.uint32(OUT_DIM)

    @plsc.parallel_loop(0, GPW * OUT_DIM, step=LANES, unroll=8)
    def _(i):
        hist[pl.ds(i, LANES)] = zeros16

    pltpu.sync_copy(ptr_hbm, ptr_v)

    pw = ptr_v[pl.ds(g0, LANES)]
    wstart = pw[0]
    pe = ptr_v[pl.ds(g0 + GPW, LANES)]
    wend = pe[0]
    wn = wend - wstart
    base_a = (wstart // 8) * 8
    nch = (wn + (CHUNK - 1)) // CHUNK

    def dma_start(c, buf, sem):
        a = jnp.minimum(base_a + c * CHUNK, total - (CHUNK + 8))
        pltpu.async_copy(x_hbm.at[pl.ds(a, CHUNK + 8)], buf.at[pl.ds(0, CHUNK + 8)], sem)

    def dma_wait(buf, sem):
        pltpu.make_async_copy(x_hbm.at[pl.ds(0, CHUNK + 8)],
                              buf.at[pl.ds(0, CHUNK + 8)], sem).wait()

    def scat(colors, mask, bidx):
        idx = bidx + colors
        plsc.addupdate_scatter(hist, [idx], ones16, mask=mask)

    def process(c, buf, g):
        cs = wstart + c * CHUNK
        a = jnp.minimum(base_a + c * CHUNK, total - (CHUNK + 8))
        off = cs - a
        npc = jnp.minimum(wn - c * CHUNK, CHUNK)
        ce = cs + npc

        def piece_cond(st):
            p, _ = st
            return p < ce

        def piece(st):
            p, g = st
            pv = ptr_v[pl.ds(g0 + g + 1, LANES)]
            gend = pv[0]
            e = jnp.minimum(gend, ce)
            n = e - p
            bidx = g * OUT_DIM
            boff = off + (p - cs)
            nfull = n // LANES

            @plsc.parallel_loop(0, nfull, unroll=UNROLL)
            def _(v):
                colors = buf[pl.ds(boff + v * LANES, LANES)]
                mask = plsc.bitcast(colors, jnp.uint32) < udim
                scat(colors, mask, bidx)

            rem = n - nfull * LANES

            @pl.when(rem > 0)
            def _():
                colors = buf[pl.ds(boff + nfull * LANES, LANES)]
                mask = (plsc.bitcast(colors, jnp.uint32) < udim) & (iota16 < rem)
                scat(colors, mask, bidx)

            g = jnp.where(gend <= ce, g + 1, g)
            return (e, g)

        _, g = lax.while_loop(piece_cond, piece, (p := cs, g))
        return g

    @pl.when(nch > 0)
    def _():
        dma_start(jnp.int32(0), buf0, sem0)

    def pair(i, g):
        c = 2 * i

        @pl.when(c + 1 < nch)
        def _():
            dma_start(c + 1, buf1, sem1)

        dma_wait(buf0, sem0)
        g = process(c, buf0, g)

        def second(g):
            @pl.when(c + 2 < nch)
            def _():
                dma_start(c + 2, buf0, sem0)

            dma_wait(buf1, sem1)
            return process(c + 1, buf1, g)

        return lax.cond(c + 1 < nch, second, lambda g: g, g)

    lax.fori_loop(0, (nch + 1) // 2, pair, jnp.int32(0))

    for g in range(GPW):
        pltpu.async_copy(hist.at[pl.ds(g * OUT_DIM, OUT_DIM)],
                         out_hbm.at[g0 + g], wsem)
    for g in range(GPW):
        pltpu.make_async_copy(hist.at[pl.ds(g * OUT_DIM, OUT_DIM)],
                              out_hbm.at[g0 + g], wsem).wait()


def kernel(x, ptr):
    x32 = x.astype(jnp.int32)
    ptr32 = ptr.astype(jnp.int32)
    pad = jnp.full((PTR_PAD - ptr32.shape[0],), x32.shape[0], jnp.int32)
    ptrp = jnp.concatenate([ptr32, pad])
    return _count_kernel(x32, ptrp)
